# baseline (speedup 1.0000x reference)
import jax
import jax.numpy as jnp
from jax import lax
from jax.experimental import pallas as pl
from jax.experimental.pallas import tpu as pltpu
from jax.experimental.pallas import tpu_sc as plsc

N_STATES = 1000000
N_ACTIONS = 16
BATCH = 16384

_NC = 2
_NS = 16
_NW = _NC * _NS
_BPW = BATCH // _NW
_CHUNK = 128
_NCHUNK = _BPW // _CHUNK


def _gather_kernel(idx_hbm, table_hbm, out_hbm, idx_v, rows_v, sem):
    wid = lax.axis_index("s") * _NC + lax.axis_index("c")
    pltpu.sync_copy(idx_hbm.at[pl.ds(wid * _NCHUNK, _NCHUNK)], idx_v)
    copies = [
        pltpu.async_copy(
            table_hbm.at[idx_v.at[j]],
            rows_v.at[pl.ds(j * _CHUNK, _CHUNK)],
            sem,
        )
        for j in range(_NCHUNK)
    ]
    for c in copies:
        c.wait()
    pltpu.sync_copy(rows_v, out_hbm.at[pl.ds(wid * _BPW, _BPW)])


@jax.jit
def kernel(state, policy):
    idx2d = jnp.asarray(state, jnp.int32).reshape(_NW * _NCHUNK, _CHUNK)
    run = pl.kernel(
        _gather_kernel,
        out_type=jax.ShapeDtypeStruct((BATCH, N_ACTIONS), jnp.float32),
        mesh=plsc.VectorSubcoreMesh(core_axis_name="c", subcore_axis_name="s"),
        scratch_types=[
            pltpu.VMEM((_NCHUNK, _CHUNK), jnp.int32),
            pltpu.VMEM((_BPW, N_ACTIONS), jnp.float32),
            pltpu.SemaphoreType.DMA,
        ],
        compiler_params=pltpu.CompilerParams(use_tc_tiling_on_sc=False),
    )
    return run(idx2d, policy)


# speedup vs baseline: 5.2776x; 5.2776x over previous
import jax
import jax.numpy as jnp
from jax import lax
from jax.experimental import pallas as pl
from jax.experimental.pallas import tpu as pltpu
from jax.experimental.pallas import tpu_sc as plsc

N_STATES = 1000000
N_ACTIONS = 16
BATCH = 16384

_NC = 2
_NS = 16
_NW = _NC * _NS
_BPW = BATCH // _NW
_CH = 16
_NCHUNK = _BPW // _CH
_NTC = 999936 // 128


def _gather_kernel(state_hbm, tableT_hbm, outT_hbm,
                   idx_v, tix_v, ring, stage, sems):
    wid = lax.axis_index("s") * _NC + lax.axis_index("c")
    base = wid * _BPW
    pltpu.sync_copy(state_hbm.at[pl.ds(base, _BPW)], idx_v)

    iota = lax.iota(jnp.int32, 16)

    def fire(c, p):
        handles = []
        t16 = (idx_v[pl.ds(c * _CH, _CH)] >> 7) * 128
        for k in range(_CH):
            tcol = pl.multiple_of(t16[k], 128)
            handles.append(pltpu.async_copy(
                tableT_hbm.at[pl.ds(0, 8), pl.ds(tcol, 128)],
                ring.at[p, k], sems.at[p]))
            handles.append(pltpu.async_copy(
                tableT_hbm.at[pl.ds(8, 8), pl.ds(tcol, 128)],
                ring.at[p, _CH + k], sems.at[p]))
        return handles

    def process(c, p):
        mv = idx_v[pl.ds(c * _CH, _CH)] & 127
        g = (c * _CH) >> 7
        lc = (c * _CH) & 127
        for tr in range(2):
            for f in range(8):
                v = plsc.load_gather(
                    ring.at[p],
                    [iota + tr * _CH, jnp.full((16,), f, jnp.int32), mv])
                plsc.store_scatter(
                    stage,
                    [jnp.full((16,), tr * 4 + g, jnp.int32),
                     jnp.full((16,), f, jnp.int32),
                     lc + iota],
                    v)

    handles = fire(0, 0)
    for c in range(_NCHUNK):
        p = c % 2
        nxt = fire(c + 1, 1 - p) if c + 1 < _NCHUNK else None
        for h in handles:
            h.wait()
        process(c, p)
        handles = nxt

    for k in range(8):
        pltpu.sync_copy(
            stage.at[k],
            outT_hbm.at[pl.ds((k // 4) * 8, 8),
                        pl.ds(base + (k % 4) * 128, 128)])


@jax.jit
def kernel(state, policy):
    run = pl.kernel(
        _gather_kernel,
        out_type=jax.ShapeDtypeStruct((N_ACTIONS, BATCH), jnp.float32),
        mesh=plsc.VectorSubcoreMesh(core_axis_name="c", subcore_axis_name="s"),
        scratch_types=[
            pltpu.VMEM((_BPW,), jnp.int32),
            pltpu.VMEM((_BPW,), jnp.int32),
            pltpu.VMEM((2, 2 * _CH, 8, 128), jnp.float32),
            pltpu.VMEM((8, 8, 128), jnp.float32),
            pltpu.SemaphoreType.DMA((2,)),
        ],
        compiler_params=pltpu.CompilerParams(
            disable_bounds_checks=True, needs_layout_passes=False
        ),
    )
    outT = run(jnp.asarray(state, jnp.int32), jnp.transpose(policy))
    return jnp.transpose(outT)


# speedup vs baseline: 5.7826x; 1.0957x over previous
import jax
import jax.numpy as jnp
from jax import lax
from jax.experimental import pallas as pl
from jax.experimental.pallas import tpu as pltpu
from jax.experimental.pallas import tpu_sc as plsc

N_STATES = 1000000
N_ACTIONS = 16
BATCH = 16384

_NC = 2
_NS = 16
_NW = _NC * _NS
_BPW = BATCH // _NW
_CH = 16
_NCHUNK = _BPW // _CH
_DEPTH = 3


def _gather_kernel(state_hbm, tableT_hbm, outT_hbm,
                   idx_v, ring, stage, sems):
    wid = lax.axis_index("s") * _NC + lax.axis_index("c")
    base = wid * _BPW
    pltpu.sync_copy(state_hbm.at[pl.ds(base, _BPW)], idx_v)

    iota = lax.iota(jnp.int32, 16)

    def fire(c, p):
        handles = []
        t16 = (idx_v[pl.ds(c * _CH, _CH)] >> 7) * 128
        for k in range(_CH):
            tcol = pl.multiple_of(t16[k], 128)
            handles.append(pltpu.async_copy(
                tableT_hbm.at[:, pl.ds(tcol, 128)],
                ring.at[p, k], sems.at[p]))
        return handles

    def process(c, p):
        mv = idx_v[pl.ds(c * _CH, _CH)] & 127
        g = (c * _CH) >> 7
        lc = (c * _CH) & 127
        for f in range(16):
            v = plsc.load_gather(
                ring.at[p], [iota, jnp.full((16,), f, jnp.int32), mv])
            plsc.store_scatter(
                stage,
                [jnp.full((16,), (f // 8) * 4 + g, jnp.int32),
                 jnp.full((16,), f % 8, jnp.int32),
                 lc + iota],
                v)

    inflight = [fire(c, c) for c in range(_DEPTH - 1)]
    for c in range(_NCHUNK):
        p = c % _DEPTH
        if c + _DEPTH - 1 < _NCHUNK:
            inflight.append(fire(c + _DEPTH - 1, (c + _DEPTH - 1) % _DEPTH))
        handles = inflight.pop(0)
        for h in handles:
            h.wait()
        process(c, p)

    for k in range(8):
        pltpu.sync_copy(
            stage.at[k],
            outT_hbm.at[pl.ds((k // 4) * 8, 8),
                        pl.ds(base + (k % 4) * 128, 128)])


@jax.jit
def kernel(state, policy):
    run = pl.kernel(
        _gather_kernel,
        out_type=jax.ShapeDtypeStruct((N_ACTIONS, BATCH), jnp.float32),
        mesh=plsc.VectorSubcoreMesh(core_axis_name="c", subcore_axis_name="s"),
        scratch_types=[
            pltpu.VMEM((_BPW,), jnp.int32),
            pltpu.VMEM((_DEPTH, _CH, 16, 128), jnp.float32),
            pltpu.VMEM((8, 8, 128), jnp.float32),
            pltpu.SemaphoreType.DMA((_DEPTH,)),
        ],
        compiler_params=pltpu.CompilerParams(
            disable_bounds_checks=True, needs_layout_passes=False
        ),
    )
    outT = run(jnp.asarray(state, jnp.int32), jnp.transpose(policy))
    return jnp.transpose(outT)


# speedup vs baseline: 5.7985x; 1.0027x over previous
import jax
import jax.numpy as jnp
from jax import lax
from jax.experimental import pallas as pl
from jax.experimental.pallas import tpu as pltpu
from jax.experimental.pallas import tpu_sc as plsc

N_STATES = 1000000
N_ACTIONS = 16
BATCH = 16384

_NC = 2
_NS = 16
_NW = _NC * _NS
_BPW = BATCH // _NW
_CH = 16
_NCHUNK = _BPW // _CH
_DEPTH = 3


def _gather_kernel(state_hbm, tableT_hbm, outT_hbm,
                   idx_v, ring, stage, sems):
    wid = lax.axis_index("s") * _NC + lax.axis_index("c")
    base = wid * _BPW
    pltpu.sync_copy(state_hbm.at[pl.ds(base, _BPW)], idx_v)

    iota = lax.iota(jnp.int32, 16)

    def fire(c, p):
        handles = []
        t16 = (idx_v[pl.ds(c * _CH, _CH)] >> 7) * 128
        for k in range(_CH):
            tcol = pl.multiple_of(t16[k], 128)
            handles.append(pltpu.async_copy(
                tableT_hbm.at[:, pl.ds(tcol, 128)],
                ring.at[p, k], sems.at[p]))
        return handles

    def process(c, p):
        mv = idx_v[pl.ds(c * _CH, _CH)] & 127
        g = (c * _CH) >> 7
        lc = (c * _CH) & 127
        for f in range(16):
            v = plsc.load_gather(
                ring.at[p], [iota, jnp.full((16,), f, jnp.int32), mv])
            stage[(f // 8) * 4 + g, f % 8, pl.ds(lc, 16)] = v

    inflight = [fire(c, c) for c in range(_DEPTH - 1)]
    for c in range(_NCHUNK):
        p = c % _DEPTH
        if c + _DEPTH - 1 < _NCHUNK:
            inflight.append(fire(c + _DEPTH - 1, (c + _DEPTH - 1) % _DEPTH))
        handles = inflight.pop(0)
        for h in handles:
            h.wait()
        process(c, p)

    for k in range(8):
        pltpu.sync_copy(
            stage.at[k],
            outT_hbm.at[pl.ds((k // 4) * 8, 8),
                        pl.ds(base + (k % 4) * 128, 128)])


@jax.jit
def kernel(state, policy):
    run = pl.kernel(
        _gather_kernel,
        out_type=jax.ShapeDtypeStruct((N_ACTIONS, BATCH), jnp.float32),
        mesh=plsc.VectorSubcoreMesh(core_axis_name="c", subcore_axis_name="s"),
        scratch_types=[
            pltpu.VMEM((_BPW,), jnp.int32),
            pltpu.VMEM((_DEPTH, _CH, 16, 128), jnp.float32),
            pltpu.VMEM((8, 8, 128), jnp.float32),
            pltpu.SemaphoreType.DMA((_DEPTH,)),
        ],
        compiler_params=pltpu.CompilerParams(
            disable_bounds_checks=True, needs_layout_passes=False
        ),
    )
    outT = run(jnp.asarray(state, jnp.int32), jnp.transpose(policy))
    return jnp.transpose(outT)
